# baseline (speedup 1.0000x reference)
import functools

import jax
import jax.numpy as jnp
from jax import lax
from jax.experimental import pallas as pl
from jax.experimental.pallas import tpu as pltpu
from jax.experimental.pallas import tpu_sc as plsc

N_BINS = 10
N = 8388608
NC = 2
NS = 16
NW = NC * NS
L = 16
PER_W = N // NW
CH = 16384
NCH = PER_W // CH
VEC_PER_CH = CH // L
NBANK = 4
ACC = NBANK * L * 16


def _sc_histogram(u_hbm, e_hbm, out_hbm, u_v, e_v, cnt_v, su_v, se_v, res_v):
    wid = lax.axis_index("s") * NC + lax.axis_index("c")
    zero16 = jnp.zeros((L,), jnp.float32)
    ones = jnp.ones((L,), jnp.float32)
    lane16 = lax.iota(jnp.int32, L) * 16

    def zbody(k, _):
        sl = pl.ds(k * L, L)
        cnt_v[sl] = zero16
        su_v[sl] = zero16
        se_v[sl] = zero16
        return 0

    lax.fori_loop(0, ACC // L, zbody, 0)

    def chunk_body(c, _):
        base = wid * PER_W + c * CH
        pltpu.sync_copy(u_hbm.at[pl.ds(base, CH)], u_v)
        pltpu.sync_copy(e_hbm.at[pl.ds(base, CH)], e_v)

        def vec_body(i, _):
            u = u_v[pl.ds(i * L, L)]
            e = e_v[pl.ds(i * L, L)]
            b = jnp.minimum((u * 10.0).astype(jnp.int32), 9)
            m = u > 0.0
            bank = lax.rem(i, NBANK)
            idx = lane16 + b + bank * (L * 16)
            plsc.addupdate_scatter(cnt_v, [idx], ones, mask=m)
            plsc.addupdate_scatter(su_v, [idx], u, mask=m)
            plsc.addupdate_scatter(se_v, [idx], e, mask=m)
            return 0

        lax.fori_loop(0, VEC_PER_CH, vec_body, 0)
        return 0

    lax.fori_loop(0, NCH, chunk_body, 0)

    def red(j, carry):
        c0, s0, e0 = carry
        sl = pl.ds(j * L, L)
        return (c0 + cnt_v[sl], s0 + su_v[sl], e0 + se_v[sl])

    tc_, ts_, te_ = lax.fori_loop(0, ACC // L, red,
                                  (zero16, zero16, zero16))
    res_v[pl.ds(0, L)] = tc_
    res_v[pl.ds(16, L)] = ts_
    res_v[pl.ds(32, L)] = te_
    pltpu.sync_copy(res_v, out_hbm.at[wid])


def kernel(uncertainties, errors):
    mesh = plsc.VectorSubcoreMesh(core_axis_name="c", subcore_axis_name="s")
    hist = pl.kernel(
        _sc_histogram,
        out_type=jax.ShapeDtypeStruct((NW, 48), jnp.float32),
        mesh=mesh,
        compiler_params=pltpu.CompilerParams(needs_layout_passes=False),
        scratch_types=[
            pltpu.VMEM((CH,), jnp.float32),
            pltpu.VMEM((CH,), jnp.float32),
            pltpu.VMEM((ACC,), jnp.float32),
            pltpu.VMEM((ACC,), jnp.float32),
            pltpu.VMEM((ACC,), jnp.float32),
            pltpu.VMEM((48,), jnp.float32),
        ],
    )
    partials = hist(uncertainties, errors)
    s = jnp.sum(partials, axis=0)
    cnt = s[0:N_BINS]
    su = s[16:16 + N_BINS]
    se = s[32:32 + N_BINS]
    safe = jnp.maximum(cnt, 1.0)
    contrib = jnp.abs(su / safe - se / safe) * (cnt / N)
    uce = jnp.sum(jnp.where(cnt > 0, contrib, 0.0), dtype=jnp.float32)
    return jnp.reshape(uce, (1,))


# speedup vs baseline: 2.1969x; 2.1969x over previous
import jax
import jax.numpy as jnp
from jax import lax
from jax.experimental import pallas as pl
from jax.experimental.pallas import tpu as pltpu
from jax.experimental.pallas import tpu_sc as plsc

N_BINS = 10
N = 8388608
NC = 2
NS = 16
NW = NC * NS
L = 16
PER_W = N // NW
CH = 16384
NCH = PER_W // CH
VEC_PER_CH = CH // L
NBANK = 4
UNROLL = 8
BANKSZ = L * 16


def _sc_histogram(u_hbm, e_hbm, out_hbm, u_v, e_v, res_v, sem0, sem1,
                  *accs):
    cnt_b = accs[0:NBANK]
    su_b = accs[NBANK:2 * NBANK]
    se_b = accs[2 * NBANK:3 * NBANK]
    wid = lax.axis_index("s") * NC + lax.axis_index("c")
    zero16 = jnp.zeros((L,), jnp.float32)
    ones = jnp.ones((L,), jnp.float32)
    lane16 = lax.iota(jnp.int32, L) * 16
    sems = (sem0, sem1)

    def zbody(k, _):
        sl = pl.ds(k * L, L)
        for r in accs:
            r[sl] = zero16
        return 0

    lax.fori_loop(0, BANKSZ // L, zbody, 0)

    def start(c, slot):
        base = wid * PER_W + c * CH
        du = pltpu.async_copy(u_hbm.at[pl.ds(base, CH)], u_v.at[slot],
                              sems[slot])
        de = pltpu.async_copy(e_hbm.at[pl.ds(base, CH)], e_v.at[slot],
                              sems[slot])
        return du, de

    pending = {0: start(0, 0), 1: None}
    for c in range(NCH):
        slot = c & 1
        if c + 1 < NCH:
            pending[1 - slot] = start(c + 1, 1 - slot)
        du, de = pending[slot]
        du.wait()
        de.wait()

        def gbody(g, _):
            base = g * (UNROLL * L)
            us = [u_v[slot, pl.ds(base + k * L, L)] for k in range(UNROLL)]
            es = [e_v[slot, pl.ds(base + k * L, L)] for k in range(UNROLL)]
            idxs = [lane16 + (us[k] * 10.0).astype(jnp.int32)
                    for k in range(UNROLL)]
            ms = [us[k] > 0.0 for k in range(UNROLL)]
            for k in range(UNROLL):
                bk = k % NBANK
                plsc.addupdate_scatter(cnt_b[bk], [idxs[k]], ones,
                                       mask=ms[k])
                plsc.addupdate_scatter(su_b[bk], [idxs[k]], us[k],
                                       mask=ms[k])
                plsc.addupdate_scatter(se_b[bk], [idxs[k]], es[k],
                                       mask=ms[k])
            return 0

        lax.fori_loop(0, VEC_PER_CH // UNROLL, gbody, 0)

    def red(j, carry):
        c0, s0, e0 = carry
        sl = pl.ds(j * L, L)
        for bk in range(NBANK):
            c0 = c0 + cnt_b[bk][sl]
            s0 = s0 + su_b[bk][sl]
            e0 = e0 + se_b[bk][sl]
        return (c0, s0, e0)

    tc_, ts_, te_ = lax.fori_loop(0, BANKSZ // L, red,
                                  (zero16, zero16, zero16))
    res_v[pl.ds(0, L)] = tc_
    res_v[pl.ds(16, L)] = ts_
    res_v[pl.ds(32, L)] = te_
    pltpu.sync_copy(res_v, out_hbm.at[wid])


def kernel(uncertainties, errors):
    mesh = plsc.VectorSubcoreMesh(core_axis_name="c", subcore_axis_name="s")
    hist = pl.kernel(
        _sc_histogram,
        out_type=jax.ShapeDtypeStruct((NW, 48), jnp.float32),
        mesh=mesh,
        compiler_params=pltpu.CompilerParams(needs_layout_passes=False),
        scratch_types=[
            pltpu.VMEM((2, CH), jnp.float32),
            pltpu.VMEM((2, CH), jnp.float32),
            pltpu.VMEM((48,), jnp.float32),
            pltpu.SemaphoreType.DMA,
            pltpu.SemaphoreType.DMA,
        ] + [pltpu.VMEM((BANKSZ,), jnp.float32)] * (3 * NBANK),
    )
    partials = hist(uncertainties, errors)
    s = jnp.sum(partials, axis=0)
    cnt = s[0:N_BINS]
    su = s[16:16 + N_BINS]
    se = s[32:32 + N_BINS]
    safe = jnp.maximum(cnt, 1.0)
    contrib = jnp.abs(su / safe - se / safe) * (cnt / N)
    uce = jnp.sum(jnp.where(cnt > 0, contrib, 0.0), dtype=jnp.float32)
    return jnp.reshape(uce, (1,))
